# baseline (speedup 1.0000x reference)
import functools

import jax
import jax.numpy as jnp
from jax.experimental import pallas as pl

B = 4096
N = 8
D_IN, D_H, D_LAT = 3, 64, 3
BT = 256


def _leaky(v):
    return jnp.where(v >= 0, v, 0.2 * v)


def _gatv2_tile(x_tok, Wl, bl, Wr, br, We, att, mask, edist, F):
    xl = jnp.dot(x_tok, Wl, preferred_element_type=jnp.float32) + bl
    xr = jnp.dot(x_tok, Wr, preferred_element_type=jnp.float32) + br
    xl3 = xl.reshape(BT, N, F)
    xr3 = xr.reshape(BT, N, F)
    att3 = att.reshape(1, 1, F)
    We3 = We.reshape(1, 1, F)
    cols = []
    for j in range(N):
        s = xr3 + xl3[:, j:j + 1, :] + edist[:, :, j:j + 1] * We3
        cols.append(jnp.sum(_leaky(s) * att3, axis=-1, keepdims=True))
    logits = jnp.concatenate(cols, axis=-1)
    logits = jnp.where(mask, logits, -1e9)
    m = jnp.max(logits, axis=-1, keepdims=True)
    e = jnp.exp(logits - m)
    alpha = e / jnp.sum(e, axis=-1, keepdims=True)
    out = jnp.zeros((BT, N, F), jnp.float32)
    for j in range(N):
        out = out + alpha[:, :, j:j + 1] * xl3[:, j:j + 1, :]
    return out


def _fused_kernel(x_ref,
                  enc_W1, enc_b1, enc_W2, enc_b2, enc_W3, enc_b3,
                  g1_Wl, g1_bl, g1_Wr, g1_br, g1_We, g1_att, g1_bias,
                  g2_Wl, g2_bl, g2_Wr, g2_br, g2_We, g2_att, g2_bias,
                  skip_W, skip_b, dec_W1, dec_b1, dec_W2, dec_b2,
                  dec_W3, dec_b3,
                  pre_ref, rec_ref, vis_ref, adj_ref):
    x = x_ref[...]

    node_idx = jax.lax.broadcasted_iota(jnp.int32, (BT, N), 1).astype(jnp.float32)
    pre = jnp.stack([jnp.zeros((BT, N), jnp.float32), node_idx, x], axis=-1)
    pre_ref[...] = pre

    tok = pre.reshape(BT * N, D_IN)
    h = jnp.maximum(jnp.dot(tok, enc_W1[...], preferred_element_type=jnp.float32) + enc_b1[...], 0.0)
    h = jnp.maximum(jnp.dot(h, enc_W2[...], preferred_element_type=jnp.float32) + enc_b2[...], 0.0)
    latent = jnp.dot(h, enc_W3[...], preferred_element_type=jnp.float32) + enc_b3[...]
    lat3 = latent.reshape(BT, N, D_LAT)

    mean = jnp.mean(lat3, axis=1, keepdims=True)
    vis = lat3 - mean
    var = jnp.sum(vis * vis, axis=1, keepdims=True) * (1.0 / (N - 1))
    std = jnp.sqrt(var) + 1e-8
    vis = vis / std
    vis_ref[...] = vis

    p0 = vis[:, :, 0]
    p1 = vis[:, :, 1]
    p2 = vis[:, :, 2]
    d0 = p0[:, :, None] - p0[:, None, :]
    d1 = p1[:, :, None] - p1[:, None, :]
    d2c = p2[:, :, None] - p2[:, None, :]
    dist2 = d0 * d0 + d1 * d1 + d2c * d2c
    radius_sq = 0.25 * dist2
    m0 = 0.5 * (p0[:, :, None] + p0[:, None, :])
    m1 = 0.5 * (p1[:, :, None] + p1[:, None, :])
    m2 = 0.5 * (p2[:, :, None] + p2[:, None, :])
    ii = jax.lax.broadcasted_iota(jnp.int32, (BT, N, N), 1)
    jj = jax.lax.broadcasted_iota(jnp.int32, (BT, N, N), 2)
    is_gab = jnp.ones((BT, N, N), jnp.bool_)
    for k in range(N):
        e0 = p0[:, k][:, None, None] - m0
        e1 = p1[:, k][:, None, None] - m1
        e2 = p2[:, k][:, None, None] - m2
        dk = e0 * e0 + e1 * e1 + e2 * e2
        ok = (dk >= radius_sq) | (ii == k) | (jj == k)
        is_gab = is_gab & ok
    eye = ii == jj
    adj = is_gab & (~eye)
    adj_ref[...] = adj.astype(jnp.int32)

    dist = jnp.sqrt(dist2 + 1e-20)
    adj_f = adj.astype(jnp.float32)
    num = jnp.sum(dist * adj_f, axis=(1, 2), keepdims=True)
    den = jnp.maximum(jnp.sum(adj_f, axis=(1, 2), keepdims=True), 1.0)
    mean_dist = num / den
    edist = jnp.where(eye, mean_dist, dist)
    mask = adj | eye

    a1 = _gatv2_tile(latent, g1_Wl[...], g1_bl[...], g1_Wr[...], g1_br[...],
                     g1_We[...], g1_att[...], mask, edist, D_H)
    x1 = jnp.maximum(a1 + g1_bias[...].reshape(1, 1, D_H), 0.0)

    a2 = _gatv2_tile(x1.reshape(BT * N, D_H), g2_Wl[...], g2_bl[...],
                     g2_Wr[...], g2_br[...], g2_We[...], g2_att[...],
                     mask, edist, D_LAT)
    gcn = a2 + g2_bias[...].reshape(1, 1, D_LAT)

    skip = jnp.dot(latent, skip_W[...], preferred_element_type=jnp.float32) + skip_b[...]
    comb = (gcn.reshape(BT * N, D_LAT) + skip)

    r = jnp.maximum(jnp.dot(comb, dec_W1[...], preferred_element_type=jnp.float32) + dec_b1[...], 0.0)
    r = jnp.maximum(jnp.dot(r, dec_W2[...], preferred_element_type=jnp.float32) + dec_b2[...], 0.0)
    rec = jnp.dot(r, dec_W3[...], preferred_element_type=jnp.float32) + dec_b3[...]
    rec_ref[...] = rec.reshape(BT, N, D_IN)


def _run(interpret, x, *params):
    grid = (B // BT,)

    def wspec(shape):
        nd = len(shape)
        return pl.BlockSpec(shape, lambda b: (0,) * nd)

    in_specs = [pl.BlockSpec((BT, N), lambda b: (b, 0))]
    in_specs += [wspec(p.shape) for p in params]
    out_specs = [
        pl.BlockSpec((BT, N, D_IN), lambda b: (b, 0, 0)),
        pl.BlockSpec((BT, N, D_IN), lambda b: (b, 0, 0)),
        pl.BlockSpec((BT, N, D_IN), lambda b: (b, 0, 0)),
        pl.BlockSpec((BT, N, N), lambda b: (b, 0, 0)),
    ]
    out_shape = [
        jax.ShapeDtypeStruct((B, N, D_IN), jnp.float32),
        jax.ShapeDtypeStruct((B, N, D_IN), jnp.float32),
        jax.ShapeDtypeStruct((B, N, D_IN), jnp.float32),
        jax.ShapeDtypeStruct((B, N, N), jnp.int32),
    ]
    return pl.pallas_call(
        _fused_kernel,
        grid=grid,
        in_specs=in_specs,
        out_specs=out_specs,
        out_shape=out_shape,
        interpret=interpret,
    )(x, *params)


@functools.partial(jax.jit, static_argnames=("interpret",))
def _kernel_impl(x, enc_W1, enc_b1, enc_W2, enc_b2, enc_W3, enc_b3,
                 g1_Wl, g1_bl, g1_Wr, g1_br, g1_We, g1_att, g1_bias,
                 g2_Wl, g2_bl, g2_Wr, g2_br, g2_We, g2_att, g2_bias,
                 skip_W, skip_b, dec_W1, dec_b1, dec_W2, dec_b2,
                 dec_W3, dec_b3, interpret=False):
    params = (
        enc_W1.T, enc_b1.reshape(1, -1),
        enc_W2.T, enc_b2.reshape(1, -1),
        enc_W3.T, enc_b3.reshape(1, -1),
        g1_Wl.T, g1_bl.reshape(1, -1), g1_Wr.T, g1_br.reshape(1, -1),
        g1_We.reshape(1, -1), g1_att.reshape(1, -1), g1_bias.reshape(1, -1),
        g2_Wl.T, g2_bl.reshape(1, -1), g2_Wr.T, g2_br.reshape(1, -1),
        g2_We.reshape(1, -1), g2_att.reshape(1, -1), g2_bias.reshape(1, -1),
        skip_W.T, skip_b.reshape(1, -1),
        dec_W1.T, dec_b1.reshape(1, -1),
        dec_W2.T, dec_b2.reshape(1, -1),
        dec_W3.T, dec_b3.reshape(1, -1),
    )
    pre, rec, vis, adj = _run(interpret, x, *params)
    return pre, rec, vis, adj


def kernel(x, enc_W1, enc_b1, enc_W2, enc_b2, enc_W3, enc_b3, g1_Wl, g1_bl, g1_Wr, g1_br, g1_We, g1_att, g1_bias, g2_Wl, g2_bl, g2_Wr, g2_br, g2_We, g2_att, g2_bias, skip_W, skip_b, dec_W1, dec_b1, dec_W2, dec_b2, dec_W3, dec_b3):
    return _kernel_impl(x, enc_W1, enc_b1, enc_W2, enc_b2, enc_W3, enc_b3,
                        g1_Wl, g1_bl, g1_Wr, g1_br, g1_We, g1_att, g1_bias,
                        g2_Wl, g2_bl, g2_Wr, g2_br, g2_We, g2_att, g2_bias,
                        skip_W, skip_b, dec_W1, dec_b1, dec_W2, dec_b2,
                        dec_W3, dec_b3)


# speedup vs baseline: 1.6755x; 1.6755x over previous
import functools

import jax
import jax.numpy as jnp
from jax.experimental import pallas as pl

B = 4096
N = 8
D_IN, D_H, D_LAT = 3, 64, 3
NF = N * D_H
BT = 256
R = BT * N


def _leaky(v):
    return jnp.where(v >= 0, v, 0.2 * v)


def _gat_imaj(x_tok, edist_im, SM, WlT, bl_T, blbr_T, Aatt, Qb, Psum, mask_im):
    xl = jnp.dot(x_tok, WlT, preferred_element_type=jnp.float32)
    xl8 = xl.reshape(N, BT, D_H)
    xl_flat = jnp.concatenate([xl8[j] for j in range(N)], axis=1)
    xl_b = xl_flat + bl_T
    xl_s = xl_b + blbr_T
    xin = jnp.concatenate([x_tok, edist_im], axis=1)
    spre = jnp.dot(xin, SM, preferred_element_type=jnp.float32)
    s = _leaky(spre.reshape(N, BT, NF) + xl_s[None])
    logits = jnp.dot(s.reshape(R, NF), Aatt,
                     preferred_element_type=jnp.float32)
    logits = jnp.where(mask_im, logits, -1e9)
    m = jnp.max(logits, axis=-1, keepdims=True)
    e = jnp.exp(logits - m)
    alpha = e / jnp.sum(e, axis=-1, keepdims=True)
    ab = jnp.dot(alpha, Qb, preferred_element_type=jnp.float32)
    prod = ab.reshape(N, BT, NF) * xl_b[None]
    return jnp.dot(prod.reshape(R, NF), Psum,
                   preferred_element_type=jnp.float32)


def _fused_kernel(x_ref,
                  enc_W1, enc_b1, enc_W2, enc_b2, enc_W3, enc_b3,
                  g1_SM, g1_WlT, g1_blT, g1_blbrT, g1_Aatt, g1_bias,
                  g2_SM, g2_WlT, g2_blT, g2_blbrT, g2_Aatt, g2_bias,
                  Qb, Psum, REP, EXCL, EYEF,
                  skip_W, skip_b, dec_W1, dec_b1, dec_W2, dec_b2,
                  dec_W3, dec_b3,
                  pre_ref, rec_ref, vis_ref, adj_ref):
    x = x_ref[...]

    x_im = jnp.concatenate([x[:, i:i + 1] for i in range(N)], axis=0)
    node_i = jax.lax.broadcasted_iota(jnp.int32, (N, BT, 1), 0)
    node_f = node_i.reshape(R, 1).astype(jnp.float32)
    pre_tok = jnp.concatenate(
        [jnp.zeros((R, 1), jnp.float32), node_f, x_im], axis=1)
    pre3 = pre_tok.reshape(N, BT, D_IN)
    for i in range(N):
        pre_ref[:, i, :] = pre3[i]

    h = jnp.maximum(jnp.dot(pre_tok, enc_W1[...], preferred_element_type=jnp.float32) + enc_b1[...], 0.0)
    h = jnp.maximum(jnp.dot(h, enc_W2[...], preferred_element_type=jnp.float32) + enc_b2[...], 0.0)
    latent = jnp.dot(h, enc_W3[...], preferred_element_type=jnp.float32) + enc_b3[...]
    lat_slab = latent.reshape(N, BT, D_LAT)

    mean = jnp.mean(lat_slab, axis=0, keepdims=True)
    vis0 = lat_slab - mean
    m2 = jnp.mean(vis0, axis=0, keepdims=True)
    var = jnp.mean((vis0 - m2) * (vis0 - m2), axis=0, keepdims=True) * (N / (N - 1.0))
    std = jnp.sqrt(var) + 1e-8
    vis_slab = vis0 / std
    for i in range(N):
        vis_ref[:, i, :] = vis_slab[i]

    rep = REP[...]
    excl = EXCL[...]
    eyef = EYEF[...]
    pcs, mids = [], []
    rs = jnp.zeros((BT, N * N), jnp.float32)
    dist2 = jnp.zeros((BT, N * N), jnp.float32)
    for c in range(3):
        pc = jnp.concatenate([vis_slab[n, :, c:c + 1] for n in range(N)],
                             axis=1)
        pcs.append(pc)
        pp = jnp.dot(pc, rep, preferred_element_type=jnp.float32)
        pi = pp[:, :N * N]
        pj = pp[:, N * N:]
        mid = (pi + pj) * 0.5
        mids.append(mid)
        e = pi - mid
        rs = rs + e * e
        d = pi - pj
        dist2 = dist2 + d * d
    is_gab = None
    for k in range(N):
        dk = jnp.zeros((BT, N * N), jnp.float32)
        for c in range(3):
            e = pcs[c][:, k:k + 1] - mids[c]
            dk = dk + e * e
        ok = (dk >= rs) | (excl[k:k + 1, :] > 0.5)
        is_gab = ok if is_gab is None else (is_gab & ok)
    not_eye = eyef < 0.5
    adjf = is_gab & not_eye
    dist = jnp.sqrt(dist2 + 1e-20)
    adj_flt = adjf.astype(jnp.float32)
    num = jnp.sum(dist * adj_flt, axis=1, keepdims=True)
    den = jnp.maximum(jnp.sum(adj_flt, axis=1, keepdims=True), 1.0)
    mean_dist = num / den
    edf = jnp.where(not_eye, dist, mean_dist)
    maskf = (is_gab | (~not_eye)).astype(jnp.float32)
    adjfi = adjf.astype(jnp.int32)
    for i in range(N):
        adj_ref[:, i, :] = adjfi[:, N * i:N * (i + 1)]
    edist_im = jnp.concatenate(
        [edf[:, N * i:N * (i + 1)] for i in range(N)], axis=0)
    mask_im = jnp.concatenate(
        [maskf[:, N * i:N * (i + 1)] for i in range(N)], axis=0) > 0.5

    a1 = _gat_imaj(latent, edist_im, g1_SM[...], g1_WlT[...], g1_blT[...],
                   g1_blbrT[...], g1_Aatt[...], Qb[...], Psum[...], mask_im)
    x1 = jnp.maximum(a1 + g1_bias[...], 0.0)

    a2 = _gat_imaj(x1, edist_im, g2_SM[...], g2_WlT[...], g2_blT[...],
                   g2_blbrT[...], g2_Aatt[...], Qb[...], Psum[...], mask_im)
    gcn = a2[:, :D_LAT] + g2_bias[...]

    skip = jnp.dot(latent, skip_W[...], preferred_element_type=jnp.float32) + skip_b[...]
    comb = gcn + skip

    r = jnp.maximum(jnp.dot(comb, dec_W1[...], preferred_element_type=jnp.float32) + dec_b1[...], 0.0)
    r = jnp.maximum(jnp.dot(r, dec_W2[...], preferred_element_type=jnp.float32) + dec_b2[...], 0.0)
    rec = jnp.dot(r, dec_W3[...], preferred_element_type=jnp.float32) + dec_b3[...]
    rec3 = rec.reshape(N, BT, D_IN)
    for i in range(N):
        rec_ref[:, i, :] = rec3[i]


def _run(interpret, x, *params):
    grid = (B // BT,)

    def wspec(shape):
        nd = len(shape)
        return pl.BlockSpec(shape, lambda b: (0,) * nd)

    in_specs = [pl.BlockSpec((BT, N), lambda b: (b, 0))]
    in_specs += [wspec(p.shape) for p in params]
    out_specs = [
        pl.BlockSpec((BT, N, D_IN), lambda b: (b, 0, 0)),
        pl.BlockSpec((BT, N, D_IN), lambda b: (b, 0, 0)),
        pl.BlockSpec((BT, N, D_IN), lambda b: (b, 0, 0)),
        pl.BlockSpec((BT, N, N), lambda b: (b, 0, 0)),
    ]
    out_shape = [
        jax.ShapeDtypeStruct((B, N, D_IN), jnp.float32),
        jax.ShapeDtypeStruct((B, N, D_IN), jnp.float32),
        jax.ShapeDtypeStruct((B, N, D_IN), jnp.float32),
        jax.ShapeDtypeStruct((B, N, N), jnp.int32),
    ]
    return pl.pallas_call(
        _fused_kernel,
        grid=grid,
        in_specs=in_specs,
        out_specs=out_specs,
        out_shape=out_shape,
        interpret=interpret,
    )(x, *params)


def _gat_consts(Wl, bl, Wr, br, We, att):
    F = D_H
    WlT = Wl.T
    WrT = Wr.T
    fout = WlT.shape[1]
    if fout < F:
        padw = ((0, 0), (0, F - fout))
        WlT = jnp.pad(WlT, padw)
        WrT = jnp.pad(WrT, padw)
        bl = jnp.pad(bl, (0, F - fout))
        br = jnp.pad(br, (0, F - fout))
        We = jnp.pad(We[:, 0], (0, F - fout))
        att = jnp.pad(att, (0, F - fout))
    else:
        We = We[:, 0]
    eye8 = jnp.eye(N, dtype=jnp.float32)
    WrT_tiled = jnp.tile(WrT, (1, N))
    QW = jnp.kron(eye8, We.reshape(1, F))
    SM = jnp.concatenate([WrT_tiled, QW], axis=0)
    Aatt = jnp.kron(eye8, att.reshape(F, 1))
    blT = jnp.tile(bl.reshape(1, F), (1, N))
    blbrT = jnp.tile(br.reshape(1, F), (1, N))
    return SM, WlT, blT, blbrT, Aatt


@functools.partial(jax.jit, static_argnames=("interpret",))
def _kernel_impl(x, enc_W1, enc_b1, enc_W2, enc_b2, enc_W3, enc_b3,
                 g1_Wl, g1_bl, g1_Wr, g1_br, g1_We, g1_att, g1_bias,
                 g2_Wl, g2_bl, g2_Wr, g2_br, g2_We, g2_att, g2_bias,
                 skip_W, skip_b, dec_W1, dec_b1, dec_W2, dec_b2,
                 dec_W3, dec_b3, interpret=False):
    g1c = _gat_consts(g1_Wl, g1_bl, g1_Wr, g1_br, g1_We, g1_att)
    g2c = _gat_consts(g2_Wl, g2_bl, g2_Wr, g2_br, g2_We, g2_att)
    eye8 = jnp.eye(N, dtype=jnp.float32)
    Qb = jnp.kron(eye8, jnp.ones((1, D_H), jnp.float32))
    Psum = jnp.tile(jnp.eye(D_H, dtype=jnp.float32), (N, 1))
    lane_i = jnp.arange(N * N) // N
    lane_j = jnp.arange(N * N) % N
    rows = jnp.arange(N)[:, None]
    REP = jnp.concatenate([(rows == lane_i[None, :]).astype(jnp.float32),
                           (rows == lane_j[None, :]).astype(jnp.float32)],
                          axis=1)
    EXCL = ((rows == lane_i[None, :]) | (rows == lane_j[None, :])).astype(jnp.float32)
    EYEF = (lane_i == lane_j).astype(jnp.float32).reshape(1, N * N)
    params = (
        enc_W1.T, enc_b1.reshape(1, -1),
        enc_W2.T, enc_b2.reshape(1, -1),
        enc_W3.T, enc_b3.reshape(1, -1),
        g1c[0], g1c[1], g1c[2], g1c[3], g1c[4], g1_bias.reshape(1, -1),
        g2c[0], g2c[1], g2c[2], g2c[3], g2c[4], g2_bias.reshape(1, -1),
        Qb, Psum, REP, EXCL, EYEF,
        skip_W.T, skip_b.reshape(1, -1),
        dec_W1.T, dec_b1.reshape(1, -1),
        dec_W2.T, dec_b2.reshape(1, -1),
        dec_W3.T, dec_b3.reshape(1, -1),
    )
    pre, rec, vis, adj = _run(interpret, x, *params)
    return pre, rec, vis, adj


def kernel(x, enc_W1, enc_b1, enc_W2, enc_b2, enc_W3, enc_b3, g1_Wl, g1_bl, g1_Wr, g1_br, g1_We, g1_att, g1_bias, g2_Wl, g2_bl, g2_Wr, g2_br, g2_We, g2_att, g2_bias, skip_W, skip_b, dec_W1, dec_b1, dec_W2, dec_b2, dec_W3, dec_b3):
    return _kernel_impl(x, enc_W1, enc_b1, enc_W2, enc_b2, enc_W3, enc_b3,
                        g1_Wl, g1_bl, g1_Wr, g1_br, g1_We, g1_att, g1_bias,
                        g2_Wl, g2_bl, g2_Wr, g2_br, g2_We, g2_att, g2_bias,
                        skip_W, skip_b, dec_W1, dec_b1, dec_W2, dec_b2,
                        dec_W3, dec_b3)
